# baseline (speedup 1.0000x reference)
import functools

import jax
import jax.numpy as jnp
from jax import lax
from jax.experimental import pallas as pl
from jax.experimental.pallas import tpu as pltpu
from jax.experimental.pallas import tpu_sc as plsc

PAD_IDX = 0

_NUM_CORES = 2
_NUM_SUBCORES = 16
_NW = _NUM_CORES * _NUM_SUBCORES

_CPC = 2
_NBUF = 3


def _mask_body(idx_ref, mask_ref):
    mask_ref[...] = (idx_ref[...] != PAD_IDX).astype(jnp.int32)


@functools.lru_cache(maxsize=None)
def _make_gather(batch, seq, vocab, dim):
    assert batch % _NW == 0
    rows_per_w = batch // _NW
    assert seq % _CPC == 0
    n_chunks = seq // _CPC
    chunk_rows = _CPC * rows_per_w

    mesh = plsc.VectorSubcoreMesh(
        core_axis_name="c",
        subcore_axis_name="s",
        num_cores=_NUM_CORES,
        num_subcores=_NUM_SUBCORES,
    )

    @functools.partial(
        pl.kernel,
        mesh=mesh,
        out_type=jax.ShapeDtypeStruct((seq, batch, dim), jnp.float32),
        scratch_types=[
            pltpu.VMEM((n_chunks * chunk_rows,), jnp.int32),
            pltpu.VMEM((_NBUF, chunk_rows, dim), jnp.float32),
            [pltpu.SemaphoreType.DMA] * _NBUF,
            [pltpu.SemaphoreType.DMA] * (_NBUF * _CPC),
        ],
    )
    def gather_kernel(idx_hbm, table_hbm, out_hbm, idx_v, rows_v, gsems, wsems):
        wid = lax.axis_index("s") * _NUM_CORES + lax.axis_index("c")
        row0 = wid * rows_per_w
        pltpu.sync_copy(idx_hbm.at[wid], idx_v)

        def g_copy(c):
            return pltpu.make_async_copy(
                table_hbm.at[idx_v.at[pl.ds(c * chunk_rows, chunk_rows)]],
                rows_v.at[c % _NBUF],
                gsems[c % _NBUF],
            )

        def w_copy(c, h):
            return pltpu.make_async_copy(
                rows_v.at[c % _NBUF, pl.ds(h * rows_per_w, rows_per_w)],
                out_hbm.at[c * _CPC + h].at[pl.ds(row0, rows_per_w)],
                wsems[(c % _NBUF) * _CPC + h],
            )

        for c in range(_NBUF):
            g_copy(c).start()
        for c in range(n_chunks):
            g_copy(c).wait()
            for h in range(_CPC):
                w_copy(c, h).start()
            n = c + _NBUF
            if n < n_chunks:
                for h in range(_CPC):
                    w_copy(c, h).wait()
                g_copy(n).start()
        for c in range(n_chunks - _NBUF, n_chunks):
            for h in range(_CPC):
                w_copy(c, h).wait()

    return gather_kernel


def kernel(indices, table):
    batch, seq = indices.shape
    vocab, dim = table.shape
    rows_per_w = batch // _NW
    idx3d = (
        indices.T.reshape(seq, _NW, rows_per_w)
        .transpose(1, 0, 2)
        .reshape(_NW, seq * rows_per_w)
    )

    out_sbd = _make_gather(batch, seq, vocab, dim)(idx3d, table)
    encoded = out_sbd.transpose(1, 0, 2)

    mask = pl.pallas_call(
        _mask_body,
        out_shape=jax.ShapeDtypeStruct((batch, seq), jnp.int32),
    )(indices)
    return encoded, mask


# speedup vs baseline: 1.0260x; 1.0260x over previous
import functools

import jax
import jax.numpy as jnp
from jax import lax
from jax.experimental import pallas as pl
from jax.experimental.pallas import tpu as pltpu
from jax.experimental.pallas import tpu_sc as plsc

PAD_IDX = 0

_NUM_CORES = 2
_NUM_SUBCORES = 16
_NW = _NUM_CORES * _NUM_SUBCORES

_NBUF = 5


def _mask_body(idx_ref, mask_ref):
    mask_ref[...] = (idx_ref[...] != PAD_IDX).astype(jnp.int32)


@functools.lru_cache(maxsize=None)
def _make_gather(batch, seq, vocab, dim):
    assert batch % _NW == 0
    rows_per_w = batch // _NW
    assert seq % _NBUF == 0 and seq >= 2 * _NBUF
    main_iters = seq // _NBUF - 1

    mesh = plsc.VectorSubcoreMesh(
        core_axis_name="c",
        subcore_axis_name="s",
        num_cores=_NUM_CORES,
        num_subcores=_NUM_SUBCORES,
    )

    @functools.partial(
        pl.kernel,
        mesh=mesh,
        out_type=jax.ShapeDtypeStruct((seq, batch, dim), jnp.float32),
        scratch_types=[
            pltpu.VMEM((seq, rows_per_w), jnp.int32),
            pltpu.VMEM((_NBUF, rows_per_w, dim), jnp.float32),
            [pltpu.SemaphoreType.DMA] * _NBUF,
            [pltpu.SemaphoreType.DMA] * _NBUF,
        ],
    )
    def gather_kernel(idx_hbm, table_hbm, out_hbm, idx_v, rows_v, gsems, wsems):
        wid = lax.axis_index("s") * _NUM_CORES + lax.axis_index("c")
        row0 = wid * rows_per_w
        pltpu.sync_copy(idx_hbm.at[wid], idx_v)

        def g_copy(b, c):
            return pltpu.make_async_copy(
                table_hbm.at[idx_v.at[c]], rows_v.at[b], gsems[b]
            )

        def w_copy(b, c):
            return pltpu.make_async_copy(
                rows_v.at[b],
                out_hbm.at[c].at[pl.ds(row0, rows_per_w)],
                wsems[b],
            )

        for b in range(_NBUF):
            g_copy(b, b).start()

        def body(j, carry):
            g = j * _NBUF
            for b in range(_NBUF):
                c = g + b
                g_copy(b, c).wait()
                w_copy(b, c).start()
                w_copy(b, c).wait()
                g_copy(b, c + _NBUF).start()
            return carry

        lax.fori_loop(0, main_iters, body, 0)

        tail = seq - _NBUF
        for b in range(_NBUF):
            g_copy(b, tail + b).wait()
            w_copy(b, tail + b).start()
        for b in range(_NBUF):
            w_copy(b, tail + b).wait()

    return gather_kernel


def kernel(indices, table):
    batch, seq = indices.shape
    vocab, dim = table.shape
    rows_per_w = batch // _NW
    idx3d = indices.T.reshape(seq, _NW, rows_per_w).transpose(1, 0, 2)

    out_sbd = _make_gather(batch, seq, vocab, dim)(idx3d, table)
    encoded = out_sbd.transpose(1, 0, 2)

    mask = pl.pallas_call(
        _mask_body,
        out_shape=jax.ShapeDtypeStruct((batch, seq), jnp.int32),
    )(indices)
    return encoded, mask
